# Initial kernel scaffold; baseline (speedup 1.0000x reference)
import jax
import jax.numpy as jnp
from jax.experimental import pallas as pl


def kernel(x, edge_index, W_l, b_l, W_r):
    raise NotImplementedError("write your pallas kernel here")


# speedup vs baseline: 4.8067x; 4.8067x over previous
import functools

import jax
import jax.numpy as jnp
from jax import lax
from jax.experimental import pallas as pl
from jax.experimental.pallas import tpu as pltpu
from jax.experimental.pallas import tpu_sc as plsc

N_NODES = 10000
N_EDGES = 320000
D = 128

NC = 2
NS = 16
NW = NC * NS
EPW = N_EDGES // NW
K = 80
NCHUNK = EPW // K
NBLK = N_NODES // K
BPT = (NBLK + NS - 1) // NS

_SC_MESH = dict(core_axis_name="c", subcore_axis_name="s")


def _zero_rows(rows_v):
    def zrow(i, carry):
        r = i // (D // 16)
        col = (i % (D // 16)) * 16
        rows_v[r, pl.ds(col, 16)] = jnp.zeros((16,), jnp.float32)
        return carry
    lax.fori_loop(0, K * (D // 16), zrow, 0)


def _zero_shared(rows_v, sh, s):
    def zblk(b, carry):
        blk = b * NS + s
        @pl.when(blk < NBLK)
        def _():
            pltpu.sync_copy(rows_v, sh.at[pl.ds(blk * K, K)])
        return carry
    lax.fori_loop(0, BPT, zblk, 0)


def _write_back(sh, out, c, s):
    def wblk(b, carry):
        blk = b * NS + s
        @pl.when(blk < NBLK)
        def _():
            pltpu.sync_copy(sh.at[pl.ds(blk * K, K)],
                            out.at[pl.ds(c * N_NODES + blk * K, K)])
        return carry
    lax.fori_loop(0, BPT, wblk, 0)


def _sc_agg(x, src, dst):
    mesh = plsc.VectorSubcoreMesh(**_SC_MESH)

    @functools.partial(
        pl.kernel,
        out_type=jax.ShapeDtypeStruct((NC * N_NODES, D), jnp.float32),
        mesh=mesh,
        scratch_types=[
            pltpu.VMEM((K,), jnp.int32),
            pltpu.VMEM((K,), jnp.int32),
            pltpu.VMEM((K, D), jnp.float32),
            pltpu.VMEM_SHARED((N_NODES, D), jnp.float32),
            pltpu.SemaphoreType.DMA,
        ],
    )
    def sc_kernel(x_hbm, src_hbm, dst_hbm, agg_out, src_v, dst_v, rows_v, sh,
                  sem):
        c = lax.axis_index("c")
        s = lax.axis_index("s")
        wid = c * NS + s

        _zero_rows(rows_v)
        _zero_shared(rows_v, sh, s)
        plsc.subcore_barrier()

        def chunk(i, carry):
            base = wid * EPW + i * K
            pltpu.sync_copy(src_hbm.at[pl.ds(base, K)], src_v)
            pltpu.sync_copy(dst_hbm.at[pl.ds(base, K)], dst_v)
            pltpu.async_copy(x_hbm.at[src_v], rows_v, sem).wait()
            pltpu.sync_copy(rows_v, sh.at[dst_v], add=True)
            return carry
        lax.fori_loop(0, NCHUNK, chunk, 0)

        plsc.subcore_barrier()
        _write_back(sh, agg_out, c, s)

    return sc_kernel(x, src, dst)


def _sc_cnt(dst):
    mesh = plsc.VectorSubcoreMesh(**_SC_MESH)

    @functools.partial(
        pl.kernel,
        out_type=jax.ShapeDtypeStruct((NC * N_NODES, D), jnp.float32),
        mesh=mesh,
        scratch_types=[
            pltpu.VMEM((K,), jnp.int32),
            pltpu.VMEM((K, D), jnp.float32),
            pltpu.VMEM_SHARED((N_NODES, D), jnp.float32),
        ],
    )
    def sc_kernel(dst_hbm, cnt_out, dst_v, ones_v, sh):
        c = lax.axis_index("c")
        s = lax.axis_index("s")
        wid = c * NS + s

        _zero_rows(ones_v)
        _zero_shared(ones_v, sh, s)

        def orow(i, carry):
            r = i // (D // 16)
            col = (i % (D // 16)) * 16
            ones_v[r, pl.ds(col, 16)] = jnp.ones((16,), jnp.float32)
            return carry
        lax.fori_loop(0, K * (D // 16), orow, 0)

        plsc.subcore_barrier()

        def chunk(i, carry):
            base = wid * EPW + i * K
            pltpu.sync_copy(dst_hbm.at[pl.ds(base, K)], dst_v)
            pltpu.sync_copy(ones_v, sh.at[dst_v], add=True)
            return carry
        lax.fori_loop(0, NCHUNK, chunk, 0)

        plsc.subcore_barrier()
        _write_back(sh, cnt_out, c, s)

    return sc_kernel(dst)


def _tc_finish(aparts, cparts, x, wlT, bl, wrT):
    BR = 1000
    nb = N_NODES // BR

    def body(a0_r, a1_r, c0_r, c1_r, x_r, wl_r, bl_r, wr_r, o_r):
        agg = a0_r[...] + a1_r[...]
        cnt = c0_r[:, :1] + c1_r[:, :1]
        mean = agg / jnp.maximum(cnt, 1.0)
        acc = jnp.dot(mean, wl_r[...], preferred_element_type=jnp.float32)
        acc = acc + bl_r[...]
        acc = acc + jnp.dot(x_r[...], wr_r[...],
                            preferred_element_type=jnp.float32)
        o_r[...] = jnp.where(acc > 0.0, acc, jnp.exp(acc) - 1.0)

    return pl.pallas_call(
        body,
        grid=(nb,),
        in_specs=[
            pl.BlockSpec((BR, D), lambda i: (i, 0)),
            pl.BlockSpec((BR, D), lambda i: (i + nb, 0)),
            pl.BlockSpec((BR, D), lambda i: (i, 0)),
            pl.BlockSpec((BR, D), lambda i: (i + nb, 0)),
            pl.BlockSpec((BR, D), lambda i: (i, 0)),
            pl.BlockSpec((D, D), lambda i: (0, 0)),
            pl.BlockSpec((1, D), lambda i: (0, 0)),
            pl.BlockSpec((D, D), lambda i: (0, 0)),
        ],
        out_specs=pl.BlockSpec((BR, D), lambda i: (i, 0)),
        out_shape=jax.ShapeDtypeStruct((N_NODES, D), jnp.float32),
    )(aparts, aparts, cparts, cparts, x, wlT, bl, wrT)


def kernel(x, edge_index, W_l, b_l, W_r):
    src = edge_index[0].astype(jnp.int32)
    dst = edge_index[1].astype(jnp.int32)
    aparts = _sc_agg(x, src, dst)
    cparts = _sc_cnt(dst)
    return _tc_finish(aparts, cparts, x, W_l.T, b_l.reshape(1, D), W_r.T)


# speedup vs baseline: 6.1012x; 1.2693x over previous
import functools

import jax
import jax.numpy as jnp
from jax import lax
from jax.experimental import pallas as pl
from jax.experimental.pallas import tpu as pltpu
from jax.experimental.pallas import tpu_sc as plsc

N_NODES = 10000
N_EDGES = 320000
D = 128

NC = 2
NS = 16
NW = NC * NS
EPW = N_EDGES // NW
K = 80
NCHUNK = EPW // K
NBLK = N_NODES // K
BPT = (NBLK + NS - 1) // NS

_SC_MESH = dict(core_axis_name="c", subcore_axis_name="s")


def _zero_rows(rows_v):
    def zrow(i, carry):
        r = i // (D // 16)
        col = (i % (D // 16)) * 16
        rows_v[r, pl.ds(col, 16)] = jnp.zeros((16,), jnp.float32)
        return carry
    lax.fori_loop(0, K * (D // 16), zrow, 0)


def _zero_shared(rows_v, sh, s, sem):
    def zblk(b, carry):
        blk = b * NS + s
        @pl.when(blk < NBLK)
        def _():
            pltpu.async_copy(rows_v, sh.at[pl.ds(blk * K, K)], sem)
        return carry
    lax.fori_loop(0, BPT, zblk, 0)

    def zdrain(b, carry):
        blk = b * NS + s
        @pl.when(blk < NBLK)
        def _():
            pltpu.make_async_copy(rows_v, sh.at[pl.ds(blk * K, K)], sem).wait()
        return carry
    lax.fori_loop(0, BPT, zdrain, 0)


def _write_back(sh, out, c, s, sem):
    def wblk(b, carry):
        blk = b * NS + s
        @pl.when(blk < NBLK)
        def _():
            pltpu.async_copy(sh.at[pl.ds(blk * K, K)],
                             out.at[pl.ds(c * N_NODES + blk * K, K)], sem)
        return carry
    lax.fori_loop(0, BPT, wblk, 0)

    def wdrain(b, carry):
        blk = b * NS + s
        @pl.when(blk < NBLK)
        def _():
            pltpu.make_async_copy(
                sh.at[pl.ds(blk * K, K)],
                out.at[pl.ds(c * N_NODES + blk * K, K)], sem).wait()
        return carry
    lax.fori_loop(0, BPT, wdrain, 0)


def _sc_agg(x, src, dst):
    mesh = plsc.VectorSubcoreMesh(**_SC_MESH)

    @functools.partial(
        pl.kernel,
        out_type=jax.ShapeDtypeStruct((NC * N_NODES, D), jnp.float32),
        mesh=mesh,
        scratch_types=[
            pltpu.VMEM((K,), jnp.int32),
            pltpu.VMEM((K,), jnp.int32),
            pltpu.VMEM((K,), jnp.int32),
            pltpu.VMEM((K,), jnp.int32),
            pltpu.VMEM((K, D), jnp.float32),
            pltpu.VMEM((K, D), jnp.float32),
            pltpu.VMEM_SHARED((N_NODES, D), jnp.float32),
            pltpu.SemaphoreType.DMA,
            pltpu.SemaphoreType.DMA,
            pltpu.SemaphoreType.DMA,
            pltpu.SemaphoreType.DMA,
        ],
    )
    def sc_kernel(x_hbm, src_hbm, dst_hbm, agg_out,
                  src0, src1, dst0, dst1, rows0, rows1, sh,
                  sem_g0, sem_g1, sem_s0, sem_s1):
        c = lax.axis_index("c")
        s = lax.axis_index("s")
        wid = c * NS + s
        ebase = wid * EPW

        srcb = (src0, src1)
        dstb = (dst0, dst1)
        rows = (rows0, rows1)
        sem_g = (sem_g0, sem_g1)
        sem_s = (sem_s0, sem_s1)

        _zero_rows(rows0)
        _zero_shared(rows0, sh, s, sem_g0)
        plsc.subcore_barrier()

        def load_idx(i, b):
            pltpu.sync_copy(src_hbm.at[pl.ds(ebase + i * K, K)], srcb[b])
            pltpu.sync_copy(dst_hbm.at[pl.ds(ebase + i * K, K)], dstb[b])

        def start_gather(b):
            pltpu.async_copy(x_hbm.at[srcb[b]], rows[b], sem_g[b])

        def wait_gather(b):
            pltpu.make_async_copy(x_hbm.at[srcb[b]], rows[b], sem_g[b]).wait()

        def start_scatter(b):
            pltpu.async_copy(rows[b], sh.at[dstb[b]], sem_s[b], add=True)

        def wait_scatter(b):
            pltpu.make_async_copy(rows[b], sh.at[dstb[b]], sem_s[b]).wait()

        load_idx(0, 0)
        start_gather(0)

        def step(i, b):
            wait_gather(b)
            @pl.when(i >= 1)
            def _():
                wait_scatter(1 - b)
            load_idx(i + 1, 1 - b)
            start_gather(1 - b)
            start_scatter(b)

        def pair(o, carry):
            step(2 * o, 0)
            step(2 * o + 1, 1)
            return carry
        lax.fori_loop(0, NCHUNK // 2, pair, 0)

        wait_gather(0)
        wait_scatter(1)
        start_scatter(0)
        wait_scatter(0)

        plsc.subcore_barrier()
        _write_back(sh, agg_out, c, s, sem_g0)

    return sc_kernel(x, src, dst)


def _sc_cnt(dst):
    mesh = plsc.VectorSubcoreMesh(**_SC_MESH)

    @functools.partial(
        pl.kernel,
        out_type=jax.ShapeDtypeStruct((NC * N_NODES, D), jnp.float32),
        mesh=mesh,
        scratch_types=[
            pltpu.VMEM((K,), jnp.int32),
            pltpu.VMEM((K,), jnp.int32),
            pltpu.VMEM((K, D), jnp.float32),
            pltpu.VMEM_SHARED((N_NODES, D), jnp.float32),
            pltpu.SemaphoreType.DMA,
            pltpu.SemaphoreType.DMA,
        ],
    )
    def sc_kernel(dst_hbm, cnt_out, dst0, dst1, ones_v, sh, sem_s0, sem_s1):
        c = lax.axis_index("c")
        s = lax.axis_index("s")
        wid = c * NS + s
        ebase = wid * EPW

        dstb = (dst0, dst1)
        sem_s = (sem_s0, sem_s1)

        _zero_rows(ones_v)
        _zero_shared(ones_v, sh, s, sem_s0)

        def orow(i, carry):
            r = i // (D // 16)
            col = (i % (D // 16)) * 16
            ones_v[r, pl.ds(col, 16)] = jnp.ones((16,), jnp.float32)
            return carry
        lax.fori_loop(0, K * (D // 16), orow, 0)

        plsc.subcore_barrier()

        def start_scatter(b):
            pltpu.async_copy(ones_v, sh.at[dstb[b]], sem_s[b], add=True)

        def wait_scatter(b):
            pltpu.make_async_copy(ones_v, sh.at[dstb[b]], sem_s[b]).wait()

        pltpu.sync_copy(dst_hbm.at[pl.ds(ebase, K)], dst0)

        def step(i, b):
            @pl.when(i >= 1)
            def _():
                wait_scatter(1 - b)
            start_scatter(b)
            @pl.when(i + 1 < NCHUNK)
            def _():
                pltpu.sync_copy(dst_hbm.at[pl.ds(ebase + (i + 1) * K, K)],
                                dstb[1 - b])

        def pair(o, carry):
            step(2 * o, 0)
            step(2 * o + 1, 1)
            return carry
        lax.fori_loop(0, NCHUNK // 2, pair, 0)

        wait_scatter(1)
        start_scatter(0)
        wait_scatter(0)

        plsc.subcore_barrier()
        _write_back(sh, cnt_out, c, s, sem_s0)

    return sc_kernel(dst)


def _tc_finish(aparts, cparts, x, wlT, bl, wrT):
    BR = 1000
    nb = N_NODES // BR

    def body(a0_r, a1_r, c0_r, c1_r, x_r, wl_r, bl_r, wr_r, o_r):
        agg = a0_r[...] + a1_r[...]
        cnt = c0_r[:, :1] + c1_r[:, :1]
        mean = agg / jnp.maximum(cnt, 1.0)
        acc = jnp.dot(mean, wl_r[...], preferred_element_type=jnp.float32)
        acc = acc + bl_r[...]
        acc = acc + jnp.dot(x_r[...], wr_r[...],
                            preferred_element_type=jnp.float32)
        o_r[...] = jnp.where(acc > 0.0, acc, jnp.exp(acc) - 1.0)

    return pl.pallas_call(
        body,
        grid=(nb,),
        in_specs=[
            pl.BlockSpec((BR, D), lambda i: (i, 0)),
            pl.BlockSpec((BR, D), lambda i: (i + nb, 0)),
            pl.BlockSpec((BR, D), lambda i: (i, 0)),
            pl.BlockSpec((BR, D), lambda i: (i + nb, 0)),
            pl.BlockSpec((BR, D), lambda i: (i, 0)),
            pl.BlockSpec((D, D), lambda i: (0, 0)),
            pl.BlockSpec((1, D), lambda i: (0, 0)),
            pl.BlockSpec((D, D), lambda i: (0, 0)),
        ],
        out_specs=pl.BlockSpec((BR, D), lambda i: (i, 0)),
        out_shape=jax.ShapeDtypeStruct((N_NODES, D), jnp.float32),
    )(aparts, aparts, cparts, cparts, x, wlT, bl, wrT)


def kernel(x, edge_index, W_l, b_l, W_r):
    src = edge_index[0].astype(jnp.int32)
    dst = edge_index[1].astype(jnp.int32)
    aparts = _sc_agg(x, src, dst)
    cparts = _sc_cnt(dst)
    return _tc_finish(aparts, cparts, x, W_l.T, b_l.reshape(1, D), W_r.T)


# speedup vs baseline: 9.8570x; 1.6156x over previous
import functools

import jax
import jax.numpy as jnp
from jax import lax
from jax.experimental import pallas as pl
from jax.experimental.pallas import tpu as pltpu
from jax.experimental.pallas import tpu_sc as plsc

N_NODES = 10000
N_EDGES = 320000
D = 128

NC = 2
NS = 16
NW = NC * NS
EPW = N_EDGES // NW
K = 80
NCHUNK = EPW // K
NBLK = N_NODES // K
BPT = (NBLK + NS - 1) // NS

_SC_MESH = dict(core_axis_name="c", subcore_axis_name="s")


def _zero_rows(rows_v):
    def zrow(i, carry):
        r = i // (D // 16)
        col = (i % (D // 16)) * 16
        rows_v[r, pl.ds(col, 16)] = jnp.zeros((16,), jnp.float32)
        return carry
    lax.fori_loop(0, K * (D // 16), zrow, 0)


def _zero_shared(rows_v, sh, s, sem):
    def zblk(b, carry):
        blk = b * NS + s
        @pl.when(blk < NBLK)
        def _():
            pltpu.async_copy(rows_v, sh.at[pl.ds(blk * K, K)], sem)
        return carry
    lax.fori_loop(0, BPT, zblk, 0)

    def zdrain(b, carry):
        blk = b * NS + s
        @pl.when(blk < NBLK)
        def _():
            pltpu.make_async_copy(rows_v, sh.at[pl.ds(blk * K, K)], sem).wait()
        return carry
    lax.fori_loop(0, BPT, zdrain, 0)


def _write_back(sh, out, c, s, sem):
    def wblk(b, carry):
        blk = b * NS + s
        @pl.when(blk < NBLK)
        def _():
            pltpu.async_copy(sh.at[pl.ds(blk * K, K)],
                             out.at[pl.ds(c * N_NODES + blk * K, K)], sem)
        return carry
    lax.fori_loop(0, BPT, wblk, 0)

    def wdrain(b, carry):
        blk = b * NS + s
        @pl.when(blk < NBLK)
        def _():
            pltpu.make_async_copy(
                sh.at[pl.ds(blk * K, K)],
                out.at[pl.ds(c * N_NODES + blk * K, K)], sem).wait()
        return carry
    lax.fori_loop(0, BPT, wdrain, 0)


def _fill_idx(src_hbm, dst_hbm, src_all, dst_all, ebase, sem):
    pltpu.async_copy(src_hbm.at[pl.ds(ebase, EPW)], src_all, sem)

    def fire(i, carry):
        pltpu.async_copy(dst_hbm.at[pl.ds(ebase + i * K, K)],
                         dst_all.at[i], sem)
        return carry
    lax.fori_loop(0, NCHUNK, fire, 0)


def _drain_idx(src_hbm, dst_hbm, src_all, dst_all, ebase, sem):
    pltpu.make_async_copy(src_hbm.at[pl.ds(ebase, EPW)], src_all, sem).wait()

    def drain(i, carry):
        pltpu.make_async_copy(dst_hbm.at[pl.ds(ebase + i * K, K)],
                              dst_all.at[i], sem).wait()
        return carry
    lax.fori_loop(0, NCHUNK, drain, 0)


def _sc_agg(x, src, dst):
    mesh = plsc.VectorSubcoreMesh(**_SC_MESH)

    @functools.partial(
        pl.kernel,
        out_type=jax.ShapeDtypeStruct((NC * N_NODES, D), jnp.float32),
        mesh=mesh,
        scratch_types=[
            pltpu.VMEM((EPW,), jnp.int32),
            pltpu.VMEM((NCHUNK, K), jnp.int32),
            pltpu.VMEM((K, D), jnp.float32),
            pltpu.VMEM((K, D), jnp.float32),
            pltpu.VMEM_SHARED((N_NODES, D), jnp.float32),
            pltpu.SemaphoreType.DMA,
            pltpu.SemaphoreType.DMA,
            pltpu.SemaphoreType.DMA,
            pltpu.SemaphoreType.DMA,
            pltpu.SemaphoreType.DMA,
        ],
    )
    def sc_kernel(x_hbm, src_hbm, dst_hbm, agg_out,
                  src_all, dst_all, rows0, rows1, sh,
                  sem_g0, sem_g1, sem_s0, sem_s1, sem_i):
        c = lax.axis_index("c")
        s = lax.axis_index("s")
        wid = c * NS + s
        ebase = wid * EPW

        rows = (rows0, rows1)
        sem_g = (sem_g0, sem_g1)
        sem_s = (sem_s0, sem_s1)

        _fill_idx(src_hbm, dst_hbm, src_all, dst_all, ebase, sem_i)
        _zero_rows(rows0)
        _zero_shared(rows0, sh, s, sem_g0)
        _drain_idx(src_hbm, dst_hbm, src_all, dst_all, ebase, sem_i)
        plsc.subcore_barrier()

        def start_gather(i, b):
            pltpu.async_copy(x_hbm.at[src_all.at[pl.ds(i * K, K)]],
                             rows[b], sem_g[b])

        def wait_gather(i, b):
            pltpu.make_async_copy(x_hbm.at[src_all.at[pl.ds(i * K, K)]],
                                  rows[b], sem_g[b]).wait()

        def start_scatter(i, b):
            pltpu.async_copy(rows[b], sh.at[dst_all.at[i]], sem_s[b],
                             add=True)

        def wait_scatter(i, b):
            pltpu.make_async_copy(rows[b], sh.at[dst_all.at[i]],
                                  sem_s[b]).wait()

        start_gather(0, 0)

        def step(i, b):
            @pl.when(i >= 1)
            def _():
                wait_scatter(i - 1, 1 - b)
            @pl.when(i + 1 < NCHUNK)
            def _():
                start_gather(i + 1, 1 - b)
            wait_gather(i, b)
            start_scatter(i, b)

        def pair(o, carry):
            step(2 * o, 0)
            step(2 * o + 1, 1)
            return carry
        lax.fori_loop(0, NCHUNK // 2, pair, 0)

        step(NCHUNK - 1, 0)
        wait_scatter(NCHUNK - 1, 0)

        plsc.subcore_barrier()
        _write_back(sh, agg_out, c, s, sem_g0)

    return sc_kernel(x, src, dst)


def _sc_cnt(dst):
    mesh = plsc.VectorSubcoreMesh(**_SC_MESH)

    @functools.partial(
        pl.kernel,
        out_type=jax.ShapeDtypeStruct((NC * N_NODES, D), jnp.float32),
        mesh=mesh,
        scratch_types=[
            pltpu.VMEM((NCHUNK, K), jnp.int32),
            pltpu.VMEM((K, D), jnp.float32),
            pltpu.VMEM_SHARED((N_NODES, D), jnp.float32),
            pltpu.SemaphoreType.DMA,
            pltpu.SemaphoreType.DMA,
        ],
    )
    def sc_kernel(dst_hbm, cnt_out, dst_all, ones_v, sh, sem_s, sem_i):
        c = lax.axis_index("c")
        s = lax.axis_index("s")
        wid = c * NS + s
        ebase = wid * EPW

        def fire_idx(i, carry):
            pltpu.async_copy(dst_hbm.at[pl.ds(ebase + i * K, K)],
                             dst_all.at[i], sem_i)
            return carry
        lax.fori_loop(0, NCHUNK, fire_idx, 0)

        _zero_rows(ones_v)
        _zero_shared(ones_v, sh, s, sem_s)

        def orow(i, carry):
            r = i // (D // 16)
            col = (i % (D // 16)) * 16
            ones_v[r, pl.ds(col, 16)] = jnp.ones((16,), jnp.float32)
            return carry
        lax.fori_loop(0, K * (D // 16), orow, 0)

        def drain_idx(i, carry):
            pltpu.make_async_copy(dst_hbm.at[pl.ds(ebase + i * K, K)],
                                  dst_all.at[i], sem_i).wait()
            return carry
        lax.fori_loop(0, NCHUNK, drain_idx, 0)

        plsc.subcore_barrier()

        def fire_sc(i, carry):
            pltpu.async_copy(ones_v, sh.at[dst_all.at[i]], sem_s, add=True)
            return carry
        lax.fori_loop(0, NCHUNK, fire_sc, 0)

        def drain_sc(i, carry):
            pltpu.make_async_copy(ones_v, sh.at[dst_all.at[i]], sem_s).wait()
            return carry
        lax.fori_loop(0, NCHUNK, drain_sc, 0)

        plsc.subcore_barrier()
        _write_back(sh, cnt_out, c, s, sem_s)

    return sc_kernel(dst)


def _tc_finish(aparts, cparts, x, wlT, bl, wrT):
    BR = 1000
    nb = N_NODES // BR

    def body(a0_r, a1_r, c0_r, c1_r, x_r, wl_r, bl_r, wr_r, o_r):
        agg = a0_r[...] + a1_r[...]
        cnt = c0_r[:, :1] + c1_r[:, :1]
        mean = agg / jnp.maximum(cnt, 1.0)
        acc = jnp.dot(mean, wl_r[...], preferred_element_type=jnp.float32)
        acc = acc + bl_r[...]
        acc = acc + jnp.dot(x_r[...], wr_r[...],
                            preferred_element_type=jnp.float32)
        o_r[...] = jnp.where(acc > 0.0, acc, jnp.exp(acc) - 1.0)

    return pl.pallas_call(
        body,
        grid=(nb,),
        in_specs=[
            pl.BlockSpec((BR, D), lambda i: (i, 0)),
            pl.BlockSpec((BR, D), lambda i: (i + nb, 0)),
            pl.BlockSpec((BR, D), lambda i: (i, 0)),
            pl.BlockSpec((BR, D), lambda i: (i + nb, 0)),
            pl.BlockSpec((BR, D), lambda i: (i, 0)),
            pl.BlockSpec((D, D), lambda i: (0, 0)),
            pl.BlockSpec((1, D), lambda i: (0, 0)),
            pl.BlockSpec((D, D), lambda i: (0, 0)),
        ],
        out_specs=pl.BlockSpec((BR, D), lambda i: (i, 0)),
        out_shape=jax.ShapeDtypeStruct((N_NODES, D), jnp.float32),
    )(aparts, aparts, cparts, cparts, x, wlT, bl, wrT)


def kernel(x, edge_index, W_l, b_l, W_r):
    src = edge_index[0].astype(jnp.int32)
    dst = edge_index[1].astype(jnp.int32)
    aparts = _sc_agg(x, src, dst)
    cparts = _sc_cnt(dst)
    return _tc_finish(aparts, cparts, x, W_l.T, b_l.reshape(1, D), W_r.T)


# speedup vs baseline: 9.8802x; 1.0023x over previous
import functools

import jax
import jax.numpy as jnp
from jax import lax
from jax.experimental import pallas as pl
from jax.experimental.pallas import tpu as pltpu
from jax.experimental.pallas import tpu_sc as plsc

N_NODES = 10000
N_EDGES = 320000
D = 128

NC = 2
NS = 16
NW = NC * NS
EPW = N_EDGES // NW
K = 80
NCHUNK = EPW // K
NBLK = N_NODES // K
BPT = (NBLK + NS - 1) // NS

_SC_MESH = dict(core_axis_name="c", subcore_axis_name="s")


def _zero_rows(rows_v):
    def zrow(i, carry):
        r = i // (D // 16)
        col = (i % (D // 16)) * 16
        rows_v[r, pl.ds(col, 16)] = jnp.zeros((16,), jnp.float32)
        return carry
    lax.fori_loop(0, K * (D // 16), zrow, 0)


def _zero_shared(rows_v, sh, s, sem):
    def zblk(b, carry):
        blk = b * NS + s
        @pl.when(blk < NBLK)
        def _():
            pltpu.async_copy(rows_v, sh.at[pl.ds(blk * K, K)], sem)
        return carry
    lax.fori_loop(0, BPT, zblk, 0)

    def zdrain(b, carry):
        blk = b * NS + s
        @pl.when(blk < NBLK)
        def _():
            pltpu.make_async_copy(rows_v, sh.at[pl.ds(blk * K, K)], sem).wait()
        return carry
    lax.fori_loop(0, BPT, zdrain, 0)


def _write_back(sh, out, c, s, sem):
    def wblk(b, carry):
        blk = b * NS + s
        @pl.when(blk < NBLK)
        def _():
            pltpu.async_copy(sh.at[pl.ds(blk * K, K)],
                             out.at[pl.ds(c * N_NODES + blk * K, K)], sem)
        return carry
    lax.fori_loop(0, BPT, wblk, 0)

    def wdrain(b, carry):
        blk = b * NS + s
        @pl.when(blk < NBLK)
        def _():
            pltpu.make_async_copy(
                sh.at[pl.ds(blk * K, K)],
                out.at[pl.ds(c * N_NODES + blk * K, K)], sem).wait()
        return carry
    lax.fori_loop(0, BPT, wdrain, 0)


def _fill_idx(src_hbm, dst_hbm, src_all, dst_all, ebase, sem):
    pltpu.async_copy(src_hbm.at[pl.ds(ebase, EPW)], src_all, sem)

    def fire(i, carry):
        pltpu.async_copy(dst_hbm.at[pl.ds(ebase + i * K, K)],
                         dst_all.at[i], sem)
        return carry
    lax.fori_loop(0, NCHUNK, fire, 0)


def _drain_idx(src_hbm, dst_hbm, src_all, dst_all, ebase, sem):
    pltpu.make_async_copy(src_hbm.at[pl.ds(ebase, EPW)], src_all, sem).wait()

    def drain(i, carry):
        pltpu.make_async_copy(dst_hbm.at[pl.ds(ebase + i * K, K)],
                              dst_all.at[i], sem).wait()
        return carry
    lax.fori_loop(0, NCHUNK, drain, 0)


def _sc_agg(x, src, dst):
    mesh = plsc.VectorSubcoreMesh(**_SC_MESH)

    @functools.partial(
        pl.kernel,
        out_type=jax.ShapeDtypeStruct((NC * N_NODES, D), jnp.float32),
        mesh=mesh,
        scratch_types=[
            pltpu.VMEM((EPW,), jnp.int32),
            pltpu.VMEM((NCHUNK, K), jnp.int32),
            pltpu.VMEM((K, D), jnp.float32),
            pltpu.VMEM((K, D), jnp.float32),
            pltpu.VMEM_SHARED((N_NODES, D), jnp.float32),
            pltpu.SemaphoreType.DMA,
            pltpu.SemaphoreType.DMA,
            pltpu.SemaphoreType.DMA,
            pltpu.SemaphoreType.DMA,
            pltpu.SemaphoreType.DMA,
        ],
    )
    def sc_kernel(x_hbm, src_hbm, dst_hbm, agg_out,
                  src_all, dst_all, rows0, rows1, sh,
                  sem_g0, sem_g1, sem_s0, sem_s1, sem_i):
        c = lax.axis_index("c")
        s = lax.axis_index("s")
        wid = c * NS + s
        ebase = wid * EPW

        rows = (rows0, rows1)
        sem_g = (sem_g0, sem_g1)
        sem_s = (sem_s0, sem_s1)

        _fill_idx(src_hbm, dst_hbm, src_all, dst_all, ebase, sem_i)
        _zero_rows(rows0)
        _zero_shared(rows0, sh, s, sem_g0)
        _drain_idx(src_hbm, dst_hbm, src_all, dst_all, ebase, sem_i)
        plsc.subcore_barrier()

        def start_gather(i, b):
            pltpu.async_copy(x_hbm.at[src_all.at[pl.ds(i * K, K)]],
                             rows[b], sem_g[b])

        def wait_gather(i, b):
            pltpu.make_async_copy(x_hbm.at[src_all.at[pl.ds(i * K, K)]],
                                  rows[b], sem_g[b]).wait()

        def start_scatter(i, b):
            pltpu.async_copy(rows[b], sh.at[dst_all.at[i]], sem_s[b],
                             add=True)

        def wait_scatter(i, b):
            pltpu.make_async_copy(rows[b], sh.at[dst_all.at[i]],
                                  sem_s[b]).wait()

        start_gather(0, 0)

        def step(i, b):
            @pl.when(i >= 1)
            def _():
                wait_scatter(i - 1, 1 - b)
            @pl.when(i + 1 < NCHUNK)
            def _():
                start_gather(i + 1, 1 - b)
            wait_gather(i, b)
            start_scatter(i, b)

        def pair(o, carry):
            step(2 * o, 0)
            step(2 * o + 1, 1)
            return carry
        lax.fori_loop(0, NCHUNK // 2, pair, 0)

        step(NCHUNK - 1, 0)
        wait_scatter(NCHUNK - 1, 0)

        plsc.subcore_barrier()
        _write_back(sh, agg_out, c, s, sem_g0)

    return sc_kernel(x, src, dst)


def _sc_cnt(dst):
    mesh = plsc.VectorSubcoreMesh(**_SC_MESH)

    @functools.partial(
        pl.kernel,
        out_type=jax.ShapeDtypeStruct((NC * N_NODES, D), jnp.float32),
        mesh=mesh,
        scratch_types=[
            pltpu.VMEM((NCHUNK, K), jnp.int32),
            pltpu.VMEM((K, D), jnp.float32),
            pltpu.VMEM_SHARED((N_NODES, D), jnp.float32),
            pltpu.SemaphoreType.DMA,
            pltpu.SemaphoreType.DMA,
        ],
    )
    def sc_kernel(dst_hbm, cnt_out, dst_all, ones_v, sh, sem_s, sem_i):
        c = lax.axis_index("c")
        s = lax.axis_index("s")
        wid = c * NS + s
        ebase = wid * EPW

        def fire_idx(i, carry):
            pltpu.async_copy(dst_hbm.at[pl.ds(ebase + i * K, K)],
                             dst_all.at[i], sem_i)
            return carry
        lax.fori_loop(0, NCHUNK, fire_idx, 0)

        _zero_rows(ones_v)
        _zero_shared(ones_v, sh, s, sem_s)

        def orow(i, carry):
            r = i // (D // 16)
            col = (i % (D // 16)) * 16
            ones_v[r, pl.ds(col, 16)] = jnp.ones((16,), jnp.float32)
            return carry
        lax.fori_loop(0, K * (D // 16), orow, 0)

        def drain_idx(i, carry):
            pltpu.make_async_copy(dst_hbm.at[pl.ds(ebase + i * K, K)],
                                  dst_all.at[i], sem_i).wait()
            return carry
        lax.fori_loop(0, NCHUNK, drain_idx, 0)

        plsc.subcore_barrier()

        def fire_sc(i, carry):
            pltpu.async_copy(ones_v, sh.at[dst_all.at[i]], sem_s, add=True)
            return carry
        lax.fori_loop(0, NCHUNK, fire_sc, 0)

        def drain_sc(i, carry):
            pltpu.make_async_copy(ones_v, sh.at[dst_all.at[i]], sem_s).wait()
            return carry
        lax.fori_loop(0, NCHUNK, drain_sc, 0)

        plsc.subcore_barrier()
        _write_back(sh, cnt_out, c, s, sem_s)

    return sc_kernel(dst)


def _tc_root(x, wrT, bl):
    BR = 1000
    nb = N_NODES // BR

    def body(x_r, wr_r, bl_r, o_r):
        o_r[...] = jnp.dot(x_r[...], wr_r[...],
                           preferred_element_type=jnp.float32) + bl_r[...]

    return pl.pallas_call(
        body,
        grid=(nb,),
        in_specs=[
            pl.BlockSpec((BR, D), lambda i: (i, 0)),
            pl.BlockSpec((D, D), lambda i: (0, 0)),
            pl.BlockSpec((1, D), lambda i: (0, 0)),
        ],
        out_specs=pl.BlockSpec((BR, D), lambda i: (i, 0)),
        out_shape=jax.ShapeDtypeStruct((N_NODES, D), jnp.float32),
    )(x, wrT, bl)


def _tc_finish(aparts, cparts, hr, wlT):
    BR = 1000
    nb = N_NODES // BR

    def body(a0_r, a1_r, c0_r, c1_r, hr_r, wl_r, o_r):
        agg = a0_r[...] + a1_r[...]
        cnt = c0_r[:, :1] + c1_r[:, :1]
        mean = agg / jnp.maximum(cnt, 1.0)
        acc = jnp.dot(mean, wl_r[...], preferred_element_type=jnp.float32)
        acc = acc + hr_r[...]
        o_r[...] = jnp.where(acc > 0.0, acc, jnp.exp(acc) - 1.0)

    return pl.pallas_call(
        body,
        grid=(nb,),
        in_specs=[
            pl.BlockSpec((BR, D), lambda i: (i, 0)),
            pl.BlockSpec((BR, D), lambda i: (i + nb, 0)),
            pl.BlockSpec((BR, D), lambda i: (i, 0)),
            pl.BlockSpec((BR, D), lambda i: (i + nb, 0)),
            pl.BlockSpec((BR, D), lambda i: (i, 0)),
            pl.BlockSpec((D, D), lambda i: (0, 0)),
        ],
        out_specs=pl.BlockSpec((BR, D), lambda i: (i, 0)),
        out_shape=jax.ShapeDtypeStruct((N_NODES, D), jnp.float32),
    )(aparts, aparts, cparts, cparts, hr, wlT)


def kernel(x, edge_index, W_l, b_l, W_r):
    src = edge_index[0].astype(jnp.int32)
    dst = edge_index[1].astype(jnp.int32)
    hr = _tc_root(x, W_r.T, b_l.reshape(1, D))
    aparts = _sc_agg(x, src, dst)
    cparts = _sc_cnt(dst)
    return _tc_finish(aparts, cparts, hr, W_l.T)


# speedup vs baseline: 10.9533x; 1.1086x over previous
import functools

import jax
import jax.numpy as jnp
from jax import lax
from jax.experimental import pallas as pl
from jax.experimental.pallas import tpu as pltpu
from jax.experimental.pallas import tpu_sc as plsc

N_NODES = 10000
N_EDGES = 320000
D = 128

NC = 2
NS = 16
NW = NC * NS
EPW = N_EDGES // NW
KA = 40
NCA = EPW // KA
K = 80
NCHUNK = EPW // K
NBLK = N_NODES // K
BPT = (NBLK + NS - 1) // NS

_SC_MESH = dict(core_axis_name="c", subcore_axis_name="s")


def _zero_rows(rows_v):
    kr = rows_v.shape[0]
    def zrow(i, carry):
        r = i // (D // 16)
        col = (i % (D // 16)) * 16
        rows_v[r, pl.ds(col, 16)] = jnp.zeros((16,), jnp.float32)
        return carry
    lax.fori_loop(0, kr * (D // 16), zrow, 0)


def _zero_shared(rows_v, sh, s, sem):
    kr = rows_v.shape[0]
    nblk = N_NODES // kr
    bpt = (nblk + NS - 1) // NS
    def zblk(b, carry):
        blk = b * NS + s
        @pl.when(blk < nblk)
        def _():
            pltpu.async_copy(rows_v, sh.at[pl.ds(blk * kr, kr)], sem)
        return carry
    lax.fori_loop(0, bpt, zblk, 0)

    def zdrain(b, carry):
        blk = b * NS + s
        @pl.when(blk < nblk)
        def _():
            pltpu.make_async_copy(rows_v, sh.at[pl.ds(blk * kr, kr)],
                                  sem).wait()
        return carry
    lax.fori_loop(0, bpt, zdrain, 0)


def _write_back(sh, out, c, s, sem):
    def wblk(b, carry):
        blk = b * NS + s
        @pl.when(blk < NBLK)
        def _():
            pltpu.async_copy(sh.at[pl.ds(blk * K, K)],
                             out.at[pl.ds(c * N_NODES + blk * K, K)], sem)
        return carry
    lax.fori_loop(0, BPT, wblk, 0)

    def wdrain(b, carry):
        blk = b * NS + s
        @pl.when(blk < NBLK)
        def _():
            pltpu.make_async_copy(
                sh.at[pl.ds(blk * K, K)],
                out.at[pl.ds(c * N_NODES + blk * K, K)], sem).wait()
        return carry
    lax.fori_loop(0, BPT, wdrain, 0)


def _fill_idx(src_hbm, dst_hbm, src_all, dst_all, ebase, sem):
    kc = dst_all.shape[1]
    nch = dst_all.shape[0]
    pltpu.async_copy(src_hbm.at[pl.ds(ebase, EPW)], src_all, sem)

    def fire(i, carry):
        pltpu.async_copy(dst_hbm.at[pl.ds(ebase + i * kc, kc)],
                         dst_all.at[i], sem)
        return carry
    lax.fori_loop(0, nch, fire, 0)


def _drain_idx(src_hbm, dst_hbm, src_all, dst_all, ebase, sem):
    kc = dst_all.shape[1]
    nch = dst_all.shape[0]
    pltpu.make_async_copy(src_hbm.at[pl.ds(ebase, EPW)], src_all, sem).wait()

    def drain(i, carry):
        pltpu.make_async_copy(dst_hbm.at[pl.ds(ebase + i * kc, kc)],
                              dst_all.at[i], sem).wait()
        return carry
    lax.fori_loop(0, nch, drain, 0)


def _sc_agg(x, src, dst):
    mesh = plsc.VectorSubcoreMesh(**_SC_MESH)

    @functools.partial(
        pl.kernel,
        out_type=jax.ShapeDtypeStruct((NC * N_NODES, D), jnp.float32),
        mesh=mesh,
        scratch_types=(
            [pltpu.VMEM((K,), jnp.int32)] * 4
            + [pltpu.VMEM((K,), jnp.int32)] * 4
            + [pltpu.VMEM((K, D), jnp.float32)] * 4
            + [pltpu.VMEM_SHARED((N_NODES, D), jnp.float32)]
            + [pltpu.SemaphoreType.DMA] * 12
        ),
    )
    def sc_kernel(x_hbm, src_hbm, dst_hbm, agg_out,
                  src0, src1, src2, src3, dst0, dst1, dst2, dst3,
                  rows0, rows1, rows2, rows3, sh,
                  sem_g0, sem_g1, sem_g2, sem_g3,
                  sem_s0, sem_s1, sem_s2, sem_s3,
                  sem_i0, sem_i1, sem_i2, sem_i3):
        c = lax.axis_index("c")
        s = lax.axis_index("s")
        wid = c * NS + s
        ebase = wid * EPW

        srcb = (src0, src1, src2, src3)
        dstb = (dst0, dst1, dst2, dst3)
        rows = (rows0, rows1, rows2, rows3)
        sem_g = (sem_g0, sem_g1, sem_g2, sem_g3)
        sem_s = (sem_s0, sem_s1, sem_s2, sem_s3)
        sem_i = (sem_i0, sem_i1, sem_i2, sem_i3)

        def fire_idx(i, b):
            pltpu.async_copy(src_hbm.at[pl.ds(ebase + i * K, K)], srcb[b],
                             sem_i[b])
            pltpu.async_copy(dst_hbm.at[pl.ds(ebase + i * K, K)], dstb[b],
                             sem_i[b])

        def wait_idx(i, b):
            pltpu.make_async_copy(src_hbm.at[pl.ds(ebase + i * K, K)],
                                  srcb[b], sem_i[b]).wait()
            pltpu.make_async_copy(dst_hbm.at[pl.ds(ebase + i * K, K)],
                                  dstb[b], sem_i[b]).wait()

        def start_gather(i, b):
            pltpu.async_copy(x_hbm.at[srcb[b]], rows[b], sem_g[b])

        def wait_gather(i, b):
            pltpu.make_async_copy(x_hbm.at[srcb[b]], rows[b],
                                  sem_g[b]).wait()

        def start_scatter(i, b):
            pltpu.async_copy(rows[b], sh.at[dstb[b]], sem_s[b], add=True)

        def wait_scatter(i, b):
            pltpu.make_async_copy(rows[b], sh.at[dstb[b]], sem_s[b]).wait()

        fire_idx(0, 0)
        fire_idx(1, 1)
        fire_idx(2, 2)
        _zero_rows(rows0)
        _zero_shared(rows0, sh, s, sem_g0)
        plsc.subcore_barrier()

        wait_idx(0, 0)
        start_gather(0, 0)
        wait_idx(1, 1)
        start_gather(1, 1)

        def step(i, b):
            b2 = (b + 2) % 4
            b3 = (b + 3) % 4
            @pl.when(i >= 1)
            def _():
                wait_scatter(i - 1, b3)
            @pl.when(i + 3 < NCHUNK)
            def _():
                fire_idx(i + 3, b3)
            @pl.when(i + 2 < NCHUNK)
            def _():
                wait_idx(i + 2, b2)
                start_gather(i + 2, b2)
            wait_gather(i, b)
            start_scatter(i, b)

        def quad(o, carry):
            step(4 * o, 0)
            step(4 * o + 1, 1)
            step(4 * o + 2, 2)
            step(4 * o + 3, 3)
            return carry
        lax.fori_loop(0, NCHUNK // 4, quad, 0)

        step(NCHUNK - 1, 0)
        wait_scatter(NCHUNK - 1, 0)

        plsc.subcore_barrier()
        _write_back(sh, agg_out, c, s, sem_g0)

    return sc_kernel(x, src, dst)


def _sc_cnt(dst):
    mesh = plsc.VectorSubcoreMesh(**_SC_MESH)

    @functools.partial(
        pl.kernel,
        out_type=jax.ShapeDtypeStruct((NC * N_NODES, D), jnp.float32),
        mesh=mesh,
        scratch_types=[
            pltpu.VMEM((NCHUNK, K), jnp.int32),
            pltpu.VMEM((K, D), jnp.float32),
            pltpu.VMEM_SHARED((N_NODES, D), jnp.float32),
            pltpu.SemaphoreType.DMA,
            pltpu.SemaphoreType.DMA,
        ],
    )
    def sc_kernel(dst_hbm, cnt_out, dst_all, ones_v, sh, sem_s, sem_i):
        c = lax.axis_index("c")
        s = lax.axis_index("s")
        wid = c * NS + s
        ebase = wid * EPW

        def fire_idx(i, carry):
            pltpu.async_copy(dst_hbm.at[pl.ds(ebase + i * K, K)],
                             dst_all.at[i], sem_i)
            return carry
        lax.fori_loop(0, NCHUNK, fire_idx, 0)

        _zero_rows(ones_v)
        _zero_shared(ones_v, sh, s, sem_s)

        def orow(i, carry):
            r = i // (D // 16)
            col = (i % (D // 16)) * 16
            ones_v[r, pl.ds(col, 16)] = jnp.ones((16,), jnp.float32)
            return carry
        lax.fori_loop(0, K * (D // 16), orow, 0)

        def drain_idx(i, carry):
            pltpu.make_async_copy(dst_hbm.at[pl.ds(ebase + i * K, K)],
                                  dst_all.at[i], sem_i).wait()
            return carry
        lax.fori_loop(0, NCHUNK, drain_idx, 0)

        plsc.subcore_barrier()

        def fire_sc(i, carry):
            pltpu.async_copy(ones_v, sh.at[dst_all.at[i]], sem_s, add=True)
            return carry
        lax.fori_loop(0, NCHUNK, fire_sc, 0)

        def drain_sc(i, carry):
            pltpu.make_async_copy(ones_v, sh.at[dst_all.at[i]], sem_s).wait()
            return carry
        lax.fori_loop(0, NCHUNK, drain_sc, 0)

        plsc.subcore_barrier()
        _write_back(sh, cnt_out, c, s, sem_s)

    return sc_kernel(dst)


def _tc_root(x, wrT, bl):
    BR = 1000
    nb = N_NODES // BR

    def body(x_r, wr_r, bl_r, o_r):
        o_r[...] = jnp.dot(x_r[...], wr_r[...],
                           preferred_element_type=jnp.float32) + bl_r[...]

    return pl.pallas_call(
        body,
        grid=(nb,),
        in_specs=[
            pl.BlockSpec((BR, D), lambda i: (i, 0)),
            pl.BlockSpec((D, D), lambda i: (0, 0)),
            pl.BlockSpec((1, D), lambda i: (0, 0)),
        ],
        out_specs=pl.BlockSpec((BR, D), lambda i: (i, 0)),
        out_shape=jax.ShapeDtypeStruct((N_NODES, D), jnp.float32),
    )(x, wrT, bl)


def _tc_finish(aparts, cparts, hr, wlT):
    BR = 1000
    nb = N_NODES // BR

    def body(a0_r, a1_r, c0_r, c1_r, hr_r, wl_r, o_r):
        agg = a0_r[...] + a1_r[...]
        cnt = c0_r[:, :1] + c1_r[:, :1]
        mean = agg / jnp.maximum(cnt, 1.0)
        acc = jnp.dot(mean, wl_r[...], preferred_element_type=jnp.float32)
        acc = acc + hr_r[...]
        o_r[...] = jnp.where(acc > 0.0, acc, jnp.exp(acc) - 1.0)

    return pl.pallas_call(
        body,
        grid=(nb,),
        in_specs=[
            pl.BlockSpec((BR, D), lambda i: (i, 0)),
            pl.BlockSpec((BR, D), lambda i: (i + nb, 0)),
            pl.BlockSpec((BR, D), lambda i: (i, 0)),
            pl.BlockSpec((BR, D), lambda i: (i + nb, 0)),
            pl.BlockSpec((BR, D), lambda i: (i, 0)),
            pl.BlockSpec((D, D), lambda i: (0, 0)),
        ],
        out_specs=pl.BlockSpec((BR, D), lambda i: (i, 0)),
        out_shape=jax.ShapeDtypeStruct((N_NODES, D), jnp.float32),
    )(aparts, aparts, cparts, cparts, hr, wlT)


def kernel(x, edge_index, W_l, b_l, W_r):
    src = edge_index[0].astype(jnp.int32)
    dst = edge_index[1].astype(jnp.int32)
    hr = _tc_root(x, W_r.T, b_l.reshape(1, D))
    aparts = _sc_agg(x, src, dst)
    cparts = _sc_cnt(dst)
    return _tc_finish(aparts, cparts, hr, W_l.T)


# speedup vs baseline: 11.0386x; 1.0078x over previous
import functools

import jax
import jax.numpy as jnp
from jax import lax
from jax.experimental import pallas as pl
from jax.experimental.pallas import tpu as pltpu
from jax.experimental.pallas import tpu_sc as plsc

N_NODES = 10000
N_EDGES = 320000
D = 128

NC = 2
NS = 16
NW = NC * NS
EPW = N_EDGES // NW
KA = 40
NCA = EPW // KA
K = 80
NCHUNK = EPW // K
NBLK = N_NODES // K
BPT = (NBLK + NS - 1) // NS

_SC_MESH = dict(core_axis_name="c", subcore_axis_name="s")


def _zero_rows(rows_v):
    kr = rows_v.shape[0]
    def zrow(i, carry):
        r = i // (D // 16)
        col = (i % (D // 16)) * 16
        rows_v[r, pl.ds(col, 16)] = jnp.zeros((16,), jnp.float32)
        return carry
    lax.fori_loop(0, kr * (D // 16), zrow, 0)


def _zero_shared(rows_v, sh, s, sem):
    kr = rows_v.shape[0]
    nblk = N_NODES // kr
    bpt = (nblk + NS - 1) // NS
    def zblk(b, carry):
        blk = b * NS + s
        @pl.when(blk < nblk)
        def _():
            pltpu.async_copy(rows_v, sh.at[pl.ds(blk * kr, kr)], sem)
        return carry
    lax.fori_loop(0, bpt, zblk, 0)

    def zdrain(b, carry):
        blk = b * NS + s
        @pl.when(blk < nblk)
        def _():
            pltpu.make_async_copy(rows_v, sh.at[pl.ds(blk * kr, kr)],
                                  sem).wait()
        return carry
    lax.fori_loop(0, bpt, zdrain, 0)


def _write_back(sh, out, c, s, sem):
    def wblk(b, carry):
        blk = b * NS + s
        @pl.when(blk < NBLK)
        def _():
            pltpu.async_copy(sh.at[pl.ds(blk * K, K)],
                             out.at[pl.ds(c * N_NODES + blk * K, K)], sem)
        return carry
    lax.fori_loop(0, BPT, wblk, 0)

    def wdrain(b, carry):
        blk = b * NS + s
        @pl.when(blk < NBLK)
        def _():
            pltpu.make_async_copy(
                sh.at[pl.ds(blk * K, K)],
                out.at[pl.ds(c * N_NODES + blk * K, K)], sem).wait()
        return carry
    lax.fori_loop(0, BPT, wdrain, 0)


def _sc_agg_cnt(x, src, dst):
    mesh = plsc.VectorSubcoreMesh(**_SC_MESH)

    @functools.partial(
        pl.kernel,
        out_type=[
            jax.ShapeDtypeStruct((NC * N_NODES, D), jnp.float32),
            jax.ShapeDtypeStruct((NC * N_NODES, D), jnp.float32),
        ],
        mesh=mesh,
        scratch_types=(
            [pltpu.VMEM((K,), jnp.int32)] * 4
            + [pltpu.VMEM((K,), jnp.int32)] * 4
            + [pltpu.VMEM((K, D), jnp.float32)] * 4
            + [pltpu.VMEM_SHARED((N_NODES, D), jnp.float32)]
            + [pltpu.SemaphoreType.DMA] * 12
        ),
    )
    def sc_kernel(x_hbm, src_hbm, dst_hbm, agg_out, cnt_out,
                  src0, src1, src2, src3, dst0, dst1, dst2, dst3,
                  rows0, rows1, rows2, rows3, sh,
                  sem_g0, sem_g1, sem_g2, sem_g3,
                  sem_s0, sem_s1, sem_s2, sem_s3,
                  sem_i0, sem_i1, sem_i2, sem_i3):
        c = lax.axis_index("c")
        s = lax.axis_index("s")
        wid = c * NS + s
        ebase = wid * EPW

        srcb = (src0, src1, src2, src3)
        dstb = (dst0, dst1, dst2, dst3)
        rows = (rows0, rows1, rows2, rows3)
        sem_g = (sem_g0, sem_g1, sem_g2, sem_g3)
        sem_s = (sem_s0, sem_s1, sem_s2, sem_s3)
        sem_i = (sem_i0, sem_i1, sem_i2, sem_i3)

        def fire_idx(i, b):
            pltpu.async_copy(src_hbm.at[pl.ds(ebase + i * K, K)], srcb[b],
                             sem_i[b])
            pltpu.async_copy(dst_hbm.at[pl.ds(ebase + i * K, K)], dstb[b],
                             sem_i[b])

        def wait_idx(i, b):
            pltpu.make_async_copy(src_hbm.at[pl.ds(ebase + i * K, K)],
                                  srcb[b], sem_i[b]).wait()
            pltpu.make_async_copy(dst_hbm.at[pl.ds(ebase + i * K, K)],
                                  dstb[b], sem_i[b]).wait()

        def start_gather(i, b):
            pltpu.async_copy(x_hbm.at[srcb[b]], rows[b], sem_g[b])

        def wait_gather(i, b):
            pltpu.make_async_copy(x_hbm.at[srcb[b]], rows[b],
                                  sem_g[b]).wait()

        def start_scatter(i, b):
            pltpu.async_copy(rows[b], sh.at[dstb[b]], sem_s[b], add=True)

        def wait_scatter(i, b):
            pltpu.make_async_copy(rows[b], sh.at[dstb[b]], sem_s[b]).wait()

        fire_idx(0, 0)
        fire_idx(1, 1)
        fire_idx(2, 2)
        _zero_rows(rows0)
        _zero_shared(rows0, sh, s, sem_g0)
        plsc.subcore_barrier()

        wait_idx(0, 0)
        start_gather(0, 0)
        wait_idx(1, 1)
        start_gather(1, 1)

        def step(i, b):
            b2 = (b + 2) % 4
            b3 = (b + 3) % 4
            @pl.when(i >= 1)
            def _():
                wait_scatter(i - 1, b3)
            @pl.when(i + 3 < NCHUNK)
            def _():
                fire_idx(i + 3, b3)
            @pl.when(i + 2 < NCHUNK)
            def _():
                wait_idx(i + 2, b2)
                start_gather(i + 2, b2)
            wait_gather(i, b)
            start_scatter(i, b)

        def quad(o, carry):
            step(4 * o, 0)
            step(4 * o + 1, 1)
            step(4 * o + 2, 2)
            step(4 * o + 3, 3)
            return carry
        lax.fori_loop(0, NCHUNK // 4, quad, 0)

        step(NCHUNK - 1, 0)
        wait_scatter(NCHUNK - 1, 0)

        plsc.subcore_barrier()
        _write_back(sh, agg_out, c, s, sem_g0)

        def fire_idx_d(i, b):
            pltpu.async_copy(dst_hbm.at[pl.ds(ebase + i * K, K)], dstb[b],
                             sem_i[b])

        def wait_idx_d(i, b):
            pltpu.make_async_copy(dst_hbm.at[pl.ds(ebase + i * K, K)],
                                  dstb[b], sem_i[b]).wait()

        def start_scatter_c(i, b):
            pltpu.async_copy(rows0, sh.at[dstb[b]], sem_s[b], add=True)

        def wait_scatter_c(i, b):
            pltpu.make_async_copy(rows0, sh.at[dstb[b]], sem_s[b]).wait()

        fire_idx_d(0, 0)
        fire_idx_d(1, 1)
        fire_idx_d(2, 2)

        _zero_rows(rows1)

        def orow(i, carry):
            r = i // (D // 16)
            col = (i % (D // 16)) * 16
            rows0[r, pl.ds(col, 16)] = jnp.ones((16,), jnp.float32)
            return carry
        lax.fori_loop(0, K * (D // 16), orow, 0)

        _zero_shared(rows1, sh, s, sem_g1)
        plsc.subcore_barrier()

        def step_c(i, b):
            b3 = (b + 3) % 4
            @pl.when(i >= 1)
            def _():
                wait_scatter_c(i - 1, b3)
            @pl.when(i + 3 < NCHUNK)
            def _():
                fire_idx_d(i + 3, b3)
            wait_idx_d(i, b)
            start_scatter_c(i, b)

        def quad_c(o, carry):
            step_c(4 * o, 0)
            step_c(4 * o + 1, 1)
            step_c(4 * o + 2, 2)
            step_c(4 * o + 3, 3)
            return carry
        lax.fori_loop(0, NCHUNK // 4, quad_c, 0)

        step_c(NCHUNK - 1, 0)
        wait_scatter_c(NCHUNK - 1, 0)

        plsc.subcore_barrier()
        _write_back(sh, cnt_out, c, s, sem_g0)

    return sc_kernel(x, src, dst)


def _tc_root(x, wrT, bl):
    BR = 1000
    nb = N_NODES // BR

    def body(x_r, wr_r, bl_r, o_r):
        o_r[...] = jnp.dot(x_r[...], wr_r[...],
                           preferred_element_type=jnp.float32) + bl_r[...]

    return pl.pallas_call(
        body,
        grid=(nb,),
        in_specs=[
            pl.BlockSpec((BR, D), lambda i: (i, 0)),
            pl.BlockSpec((D, D), lambda i: (0, 0)),
            pl.BlockSpec((1, D), lambda i: (0, 0)),
        ],
        out_specs=pl.BlockSpec((BR, D), lambda i: (i, 0)),
        out_shape=jax.ShapeDtypeStruct((N_NODES, D), jnp.float32),
    )(x, wrT, bl)


def _tc_finish(aparts, cparts, hr, wlT):
    BR = 1000
    nb = N_NODES // BR

    def body(a0_r, a1_r, c0_r, c1_r, hr_r, wl_r, o_r):
        agg = a0_r[...] + a1_r[...]
        cnt = c0_r[:, :1] + c1_r[:, :1]
        mean = agg / jnp.maximum(cnt, 1.0)
        acc = jnp.dot(mean, wl_r[...], preferred_element_type=jnp.float32)
        acc = acc + hr_r[...]
        o_r[...] = jnp.where(acc > 0.0, acc, jnp.exp(acc) - 1.0)

    return pl.pallas_call(
        body,
        grid=(nb,),
        in_specs=[
            pl.BlockSpec((BR, D), lambda i: (i, 0)),
            pl.BlockSpec((BR, D), lambda i: (i + nb, 0)),
            pl.BlockSpec((BR, D), lambda i: (i, 0)),
            pl.BlockSpec((BR, D), lambda i: (i + nb, 0)),
            pl.BlockSpec((BR, D), lambda i: (i, 0)),
            pl.BlockSpec((D, D), lambda i: (0, 0)),
        ],
        out_specs=pl.BlockSpec((BR, D), lambda i: (i, 0)),
        out_shape=jax.ShapeDtypeStruct((N_NODES, D), jnp.float32),
    )(aparts, aparts, cparts, cparts, hr, wlT)


def kernel(x, edge_index, W_l, b_l, W_r):
    src = edge_index[0].astype(jnp.int32)
    dst = edge_index[1].astype(jnp.int32)
    hr = _tc_root(x, W_r.T, b_l.reshape(1, D))
    aparts, cparts = _sc_agg_cnt(x, src, dst)
    return _tc_finish(aparts, cparts, hr, W_l.T)
